# baseline (device time: 25892 ns/iter reference)
import jax
import jax.numpy as jnp
from jax import lax
from jax.experimental import pallas as pl
from jax.experimental.pallas import tpu as pltpu

N_Z = 4
K = 8
OFF = 2


def _allreduce_z(partial):
    t, d = partial.shape
    rows = t // K

    def body(p_ref, out_ref, pbuf, psum, mbuf,
             s1_snd, s1_rcv, s2_snd, s2_rcv, s3_snd, s3_rcv):
        my_x = lax.axis_index("x")
        my_y = lax.axis_index("y")
        my_z = lax.axis_index("z")
        is_lo_edge = my_z == 0
        is_hi_edge = my_z == N_Z - 1
        is_edge = jnp.logical_or(is_lo_edge, is_hi_edge)
        is_mid = jnp.logical_not(is_edge)
        t1 = jnp.where(is_lo_edge, 1, 2)
        t2 = 3 - my_z
        t3 = jnp.where(my_z == 1, 0, 3)

        barrier_sem = pltpu.get_barrier_semaphore()
        left = lax.rem(my_z - 1 + N_Z, N_Z)
        right = lax.rem(my_z + 1, N_Z)

        @pl.when(my_z > 0)
        def _():
            pl.semaphore_signal(
                barrier_sem, inc=1,
                device_id=(my_x, my_y, left),
                device_id_type=pl.DeviceIdType.MESH,
            )

        @pl.when(my_z < N_Z - 1)
        def _():
            pl.semaphore_signal(
                barrier_sem, inc=1,
                device_id=(my_x, my_y, right),
                device_id_type=pl.DeviceIdType.MESH,
            )
        pl.semaphore_wait(barrier_sem, 1)

        @pl.when(is_mid)
        def _():
            pl.semaphore_wait(barrier_sem, 1)

        e1 = []
        e2 = []
        e3 = []
        for k in range(K):
            ro = pl.ds(k * rows, rows)
            e1.append(pltpu.make_async_remote_copy(
                src_ref=p_ref.at[ro, :],
                dst_ref=pbuf.at[k],
                send_sem=s1_snd.at[k],
                recv_sem=s1_rcv.at[k],
                device_id=(my_x, my_y, t1),
                device_id_type=pl.DeviceIdType.MESH,
            ))
            e2.append(pltpu.make_async_remote_copy(
                src_ref=psum.at[k],
                dst_ref=mbuf.at[k],
                send_sem=s2_snd.at[k],
                recv_sem=s2_rcv.at[k],
                device_id=(my_x, my_y, t2),
                device_id_type=pl.DeviceIdType.MESH,
            ))
            e3.append(pltpu.make_async_remote_copy(
                src_ref=out_ref.at[ro, :],
                dst_ref=out_ref.at[ro, :],
                send_sem=s3_snd.at[k],
                recv_sem=s3_rcv.at[k],
                device_id=(my_x, my_y, t3),
                device_id_type=pl.DeviceIdType.MESH,
            ))

        for k in range(K):
            @pl.when(is_edge)
            def _():
                e1[k].start()

        def mid_full(j):
            @pl.when(is_mid)
            def _():
                rj = pl.ds(j * rows, rows)
                e2[j].wait_recv()
                out_ref[rj, :] = psum[j, :, :] + mbuf[j, :, :]
                e3[j].start()

        for k in range(K):
            @pl.when(is_mid)
            def _():
                ro = pl.ds(k * rows, rows)
                e1[k].wait_recv()
                psum[k, :, :] = p_ref[ro, :] + pbuf[k, :, :]
                e2[k].start()

            if k >= OFF:
                mid_full(k - OFF)
        for j in range(K - OFF, K):
            mid_full(j)

        for k in range(K):
            @pl.when(is_edge)
            def _():
                e3[k].wait_recv()

        for k in range(K):
            @pl.when(is_edge)
            def _():
                e1[k].wait_send()

            @pl.when(is_mid)
            def _():
                e2[k].wait_send()
                e3[k].wait_send()

    return pl.pallas_call(
        body,
        out_shape=jax.ShapeDtypeStruct((t, d), partial.dtype),
        in_specs=[pl.BlockSpec(memory_space=pltpu.VMEM)],
        out_specs=pl.BlockSpec(memory_space=pltpu.VMEM),
        scratch_shapes=[
            pltpu.VMEM((K, rows, d), partial.dtype),
            pltpu.VMEM((K, rows, d), partial.dtype),
            pltpu.VMEM((K, rows, d), partial.dtype),
            pltpu.SemaphoreType.DMA((K,)),
            pltpu.SemaphoreType.DMA((K,)),
            pltpu.SemaphoreType.DMA((K,)),
            pltpu.SemaphoreType.DMA((K,)),
            pltpu.SemaphoreType.DMA((K,)),
            pltpu.SemaphoreType.DMA((K,)),
        ],
        compiler_params=pltpu.CompilerParams(collective_id=11),
    )(partial)


def kernel(ids, E):
    v_per, _ = E.shape
    z = lax.axis_index("z")
    local = ids - z * v_per
    mask = (local >= 0) & (local < v_per)
    safe = jnp.where(mask, local, 0)
    partial = jnp.where(mask[:, None], jnp.take(E, safe, axis=0), 0.0)
    return _allreduce_z(partial.astype(jnp.float32))
